# baseline (speedup 1.0000x reference)
import functools
import numpy as np
import jax
import jax.numpy as jnp
from jax import lax
from jax.experimental import pallas as pl
from jax.experimental.pallas import tpu as pltpu
from jax.experimental.pallas import tpu_sc as plsc

NC = 2
NS = 16
NW = NC * NS
LANES = 16
K = 128
EPS = 1e-5


def _deg_body(CH_A, CH_B, NPAD, dst_hbm, out_hbm, dst_v, hist):
    c = lax.axis_index("c")
    s = lax.axis_index("s")
    wid = s * NC + c
    nch = jnp.where(c == 0, CH_A, CH_B)
    zeros = jnp.zeros((LANES,), jnp.float32)

    @pl.loop(0, NPAD // LANES)
    def _zero(r):
        hist[pl.ds(r * LANES, LANES)] = zeros

    pltpu.sync_copy(dst_hbm.at[wid], dst_v)
    ones = jnp.ones((LANES,), jnp.float32)

    @pl.loop(0, nch)
    def _hist(j):
        for k in range(K // LANES):
            idx = dst_v[j, pl.ds(k * LANES, LANES)]
            plsc.addupdate_scatter(hist, [idx], ones)

    pltpu.sync_copy(hist, out_hbm.at[wid])


def _agg_body(CH_A, CH_B, RPT, h2_hbm, src_hbm, dst_hbm, out_hbm,
              src_v, dst_v, gbuf, agg_sh, sem):
    c = lax.axis_index("c")
    s = lax.axis_index("s")
    wid = s * NC + c
    nch = jnp.where(c == 0, CH_A, CH_B)
    zeros = jnp.zeros((LANES,), jnp.float32)
    H = gbuf.shape[2]

    @pl.loop(0, K)
    def _zero(r):
        for k in range(H // LANES):
            gbuf[0, r, pl.ds(k * LANES, LANES)] = zeros

    base = s * RPT
    off = 0
    while off < RPT:
        step = min(K, RPT - off)
        pltpu.sync_copy(gbuf.at[0, pl.ds(0, step)],
                        agg_sh.at[pl.ds(base + off, step)])
        off += step

    pltpu.sync_copy(src_hbm.at[wid], src_v)
    pltpu.sync_copy(dst_hbm.at[wid], dst_v)
    plsc.subcore_barrier()

    @pl.loop(0, nch)
    def _edges(j):
        pltpu.async_copy(h2_hbm.at[src_v.at[j]], gbuf.at[0], sem).wait()
        pltpu.sync_copy(gbuf.at[0], agg_sh.at[dst_v.at[j]], add=True)

    plsc.subcore_barrier()
    pltpu.sync_copy(agg_sh.at[pl.ds(base, RPT)],
                    out_hbm.at[c, pl.ds(base, RPT)])


def _h2_body(x_ref, w_ref, degp_ref, h2_ref, dinv_ref):
    deg = jnp.sum(degp_ref[...], axis=0) + 1.0
    dinv = lax.rsqrt(deg)
    h = jnp.dot(x_ref[...], w_ref[...], preferred_element_type=jnp.float32)
    h2_ref[...] = h * dinv[:, None]
    dinv_ref[...] = dinv[:, None]


def _bn_body(N, p_ref, h2_ref, dinv_ref, b_ref, gamma_ref, beta_ref, out_ref):
    pre = p_ref[0, :N, :] + p_ref[1, :N, :] + h2_ref[:N, :]
    pre = pre * dinv_ref[:N, :] + b_ref[...][None, :]
    mean = jnp.mean(pre, axis=0)
    var = jnp.mean((pre - mean[None, :]) ** 2, axis=0)
    out = (pre - mean[None, :]) * lax.rsqrt(var + EPS) * gamma_ref[...][None, :]
    out = out + beta_ref[...][None, :]
    out_ref[...] = jnp.maximum(out, 0.0)


def kernel(x, adj_t, W, b, gamma, beta):
    N, D = x.shape
    H = W.shape[1]
    E = adj_t.shape[1]

    FAST_SHARE = 0.594
    CH_A = -(-int(E * FAST_SHARE) // (NS * K))
    CH_B = -(-(E - NS * K * CH_A) // (NS * K))
    CH_B = max(CH_B, 1)
    CH = CH_A
    NPAD = -(-(N + 1) // 1024) * 1024
    RPT = NPAD // NS

    src = adj_t[0]
    dst = adj_t[1]
    sizes = [(CH_A if wid % NC == 0 else CH_B) * K for wid in range(NW)]
    cap = sum(sizes)
    assert cap > E
    perm = np.full((NW, CH * K), cap - 1, dtype=np.int32)
    off = 0
    for wid in range(NW):
        perm[wid, :sizes[wid]] = off + np.arange(sizes[wid], dtype=np.int32)
        off += sizes[wid]
    perm = jnp.asarray(perm)
    src_f = jnp.concatenate([src, jnp.zeros((cap - E,), jnp.int32)])
    dst_f = jnp.concatenate([dst, jnp.full((cap - E,), N, jnp.int32)])
    src2d = jnp.take(src_f, perm, axis=0).reshape(NW, CH, K)
    dst2d = jnp.take(dst_f, perm, axis=0).reshape(NW, CH, K)
    x_p = jnp.pad(x, ((0, NPAD - N), (0, 0)))

    mesh = plsc.VectorSubcoreMesh(core_axis_name="c", subcore_axis_name="s")

    degp = pl.kernel(
        functools.partial(_deg_body, CH_A, CH_B, NPAD),
        out_type=jax.ShapeDtypeStruct((NW, NPAD), jnp.float32),
        mesh=mesh,
        compiler_params=pltpu.CompilerParams(needs_layout_passes=False),
        scratch_types=[
            pltpu.VMEM((CH, K), jnp.int32),
            pltpu.VMEM((NPAD,), jnp.float32),
        ],
    )(dst2d)

    RB = NPAD // 8
    h2, dinv = pl.pallas_call(
        _h2_body,
        grid=(NPAD // RB,),
        in_specs=[
            pl.BlockSpec((RB, D), lambda i: (i, 0)),
            pl.BlockSpec((D, H), lambda i: (0, 0)),
            pl.BlockSpec((NW, RB), lambda i: (0, i)),
        ],
        out_specs=[
            pl.BlockSpec((RB, H), lambda i: (i, 0)),
            pl.BlockSpec((RB, 1), lambda i: (i, 0)),
        ],
        out_shape=[
            jax.ShapeDtypeStruct((NPAD, H), jnp.float32),
            jax.ShapeDtypeStruct((NPAD, 1), jnp.float32),
        ],
    )(x_p, W, degp)

    parts = pl.kernel(
        functools.partial(_agg_body, CH_A, CH_B, RPT),
        out_type=jax.ShapeDtypeStruct((NC, NPAD, H), jnp.float32),
        mesh=mesh,
        compiler_params=pltpu.CompilerParams(needs_layout_passes=False),
        scratch_types=[
            pltpu.VMEM((CH, K), jnp.int32),
            pltpu.VMEM((CH, K), jnp.int32),
            pltpu.VMEM((1, K, H), jnp.float32),
            pltpu.VMEM_SHARED((NPAD, H), jnp.float32),
            pltpu.SemaphoreType.DMA,
        ],
    )(h2, src2d, dst2d)

    out = pl.pallas_call(
        functools.partial(_bn_body, N),
        out_shape=jax.ShapeDtypeStruct((N, H), jnp.float32),
    )(parts, h2, dinv, b, gamma, beta)
    return out


# speedup vs baseline: 3.1671x; 3.1671x over previous
import functools
import numpy as np
import jax
import jax.numpy as jnp
from jax import lax
from jax.experimental import pallas as pl
from jax.experimental.pallas import tpu as pltpu
from jax.experimental.pallas import tpu_sc as plsc

NC = 2
NS = 16
NW = NC * NS
LANES = 16
K = 128
EPS = 1e-5


def _deg_body(CH_A, CH_B, NPAD, dst_hbm, out_hbm, dst_v, hist):
    c = lax.axis_index("c")
    s = lax.axis_index("s")
    wid = s * NC + c
    nch = jnp.where(c == 0, CH_A, CH_B)
    zeros = jnp.zeros((LANES,), jnp.float32)

    @pl.loop(0, NPAD // LANES)
    def _zero(r):
        hist[pl.ds(r * LANES, LANES)] = zeros

    pltpu.sync_copy(dst_hbm.at[wid], dst_v)
    ones = jnp.ones((LANES,), jnp.float32)

    @pl.loop(0, nch)
    def _hist(j):
        for k in range(K // LANES):
            idx = dst_v[j, pl.ds(k * LANES, LANES)]
            plsc.addupdate_scatter(hist, [idx], ones)

    pltpu.sync_copy(hist, out_hbm.at[wid])


def _agg_body(CH_A, CH_B, RPT, h2_hbm, src_hbm, dst_hbm, out_hbm,
              src_v, dst_v, gbuf, agg_sh, sem):
    c = lax.axis_index("c")
    s = lax.axis_index("s")
    wid = s * NC + c
    nch = jnp.where(c == 0, CH_A, CH_B)
    zeros = jnp.zeros((LANES,), jnp.float32)
    H = gbuf.shape[2]

    @pl.loop(0, K)
    def _zero(r):
        for k in range(H // LANES):
            gbuf[0, r, pl.ds(k * LANES, LANES)] = zeros

    base = s * RPT
    off = 0
    while off < RPT:
        step = min(K, RPT - off)
        pltpu.sync_copy(gbuf.at[0, pl.ds(0, step)],
                        agg_sh.at[pl.ds(base + off, step)])
        off += step

    pltpu.sync_copy(src_hbm.at[wid], src_v)
    pltpu.sync_copy(dst_hbm.at[wid], dst_v)
    plsc.subcore_barrier()

    @pl.loop(0, nch)
    def _edges(j):
        pltpu.async_copy(h2_hbm.at[src_v.at[j]], gbuf.at[0], sem).wait()
        pltpu.sync_copy(gbuf.at[0], agg_sh.at[dst_v.at[j]], add=True)

    plsc.subcore_barrier()
    pltpu.sync_copy(agg_sh.at[pl.ds(base, RPT)],
                    out_hbm.at[c, pl.ds(base, RPT)])


def _h2_body(x_ref, w_ref, degp_ref, h2_ref, dinv_ref):
    deg = jnp.sum(degp_ref[...], axis=0) + 1.0
    dinv = lax.rsqrt(deg)
    h = jnp.dot(x_ref[...], w_ref[...], preferred_element_type=jnp.float32)
    h2_ref[...] = h * dinv[:, None]
    dinv_ref[...] = dinv[:, None]


def _bn_body(N, p_ref, h2_ref, dinv_ref, b_ref, gamma_ref, beta_ref, out_ref):
    pre = p_ref[0, :N, :] + p_ref[1, :N, :] + h2_ref[:N, :]
    pre = pre * dinv_ref[:N, :] + b_ref[...][None, :]
    mean = jnp.mean(pre, axis=0)
    var = jnp.mean((pre - mean[None, :]) ** 2, axis=0)
    out = (pre - mean[None, :]) * lax.rsqrt(var + EPS) * gamma_ref[...][None, :]
    out = out + beta_ref[...][None, :]
    out_ref[...] = jnp.maximum(out, 0.0)


def kernel(x, adj_t, W, b, gamma, beta):
    N, D = x.shape
    H = W.shape[1]
    E = adj_t.shape[1]

    FAST_SHARE = 0.594
    CH_A = -(-int(E * FAST_SHARE) // (NS * K))
    CH_B = -(-(E - NS * K * CH_A) // (NS * K))
    CH_B = max(CH_B, 1)
    CH = CH_A
    NPAD = -(-(N + 1) // 1024) * 1024
    RPT = NPAD // NS

    src = adj_t[0]
    dst = adj_t[1]
    sizes = [(CH_A if wid % NC == 0 else CH_B) * K for wid in range(NW)]
    cap = sum(sizes)
    src_f = jnp.concatenate([src, jnp.zeros((cap - E,), jnp.int32)])
    dst_f = jnp.concatenate([dst, jnp.full((cap - E,), N, jnp.int32)])
    srows = []
    drows = []
    off = 0
    for wid in range(NW):
        seg_s = src_f[off:off + sizes[wid]]
        seg_d = dst_f[off:off + sizes[wid]]
        fill = CH * K - sizes[wid]
        if fill:
            seg_s = jnp.concatenate([seg_s, jnp.zeros((fill,), jnp.int32)])
            seg_d = jnp.concatenate([seg_d, jnp.full((fill,), N, jnp.int32)])
        srows.append(seg_s.reshape(CH, K))
        drows.append(seg_d.reshape(CH, K))
        off += sizes[wid]
    src2d = jnp.stack(srows)
    dst2d = jnp.stack(drows)
    x_p = jnp.pad(x, ((0, NPAD - N), (0, 0)))

    mesh = plsc.VectorSubcoreMesh(core_axis_name="c", subcore_axis_name="s")

    degp = pl.kernel(
        functools.partial(_deg_body, CH_A, CH_B, NPAD),
        out_type=jax.ShapeDtypeStruct((NW, NPAD), jnp.float32),
        mesh=mesh,
        compiler_params=pltpu.CompilerParams(needs_layout_passes=False),
        scratch_types=[
            pltpu.VMEM((CH, K), jnp.int32),
            pltpu.VMEM((NPAD,), jnp.float32),
        ],
    )(dst2d)

    RB = NPAD // 8
    h2, dinv = pl.pallas_call(
        _h2_body,
        grid=(NPAD // RB,),
        in_specs=[
            pl.BlockSpec((RB, D), lambda i: (i, 0)),
            pl.BlockSpec((D, H), lambda i: (0, 0)),
            pl.BlockSpec((NW, RB), lambda i: (0, i)),
        ],
        out_specs=[
            pl.BlockSpec((RB, H), lambda i: (i, 0)),
            pl.BlockSpec((RB, 1), lambda i: (i, 0)),
        ],
        out_shape=[
            jax.ShapeDtypeStruct((NPAD, H), jnp.float32),
            jax.ShapeDtypeStruct((NPAD, 1), jnp.float32),
        ],
    )(x_p, W, degp)

    parts = pl.kernel(
        functools.partial(_agg_body, CH_A, CH_B, RPT),
        out_type=jax.ShapeDtypeStruct((NC, NPAD, H), jnp.float32),
        mesh=mesh,
        compiler_params=pltpu.CompilerParams(needs_layout_passes=False),
        scratch_types=[
            pltpu.VMEM((CH, K), jnp.int32),
            pltpu.VMEM((CH, K), jnp.int32),
            pltpu.VMEM((1, K, H), jnp.float32),
            pltpu.VMEM_SHARED((NPAD, H), jnp.float32),
            pltpu.SemaphoreType.DMA,
        ],
    )(h2, src2d, dst2d)

    out = pl.pallas_call(
        functools.partial(_bn_body, N),
        out_shape=jax.ShapeDtypeStruct((N, H), jnp.float32),
    )(parts, h2, dinv, b, gamma, beta)
    return out
